# Initial kernel scaffold; baseline (speedup 1.0000x reference)
import jax
import jax.numpy as jnp
from jax.experimental import pallas as pl


def kernel(name_idx, collaborative_idx, track_uri_can_idx, artist_name_pl_idx, track_uri_pl_idx, track_name_pl_idx, duration_ms_pl_idx, album_name_pl_idx, artist_pop_pl_idx, artists_followers_pl_idx, track_pop_pl_idx, artist_genres_pl_idx, name_table, collab_table, track_uri_can_table, artist_name_table, track_uri_pl_table, track_name_table, duration_table, album_table, artist_pop_table, followers_table, track_pop_table, genres_table, W1, b1, W2, b2, W3, b3):
    raise NotImplementedError("write your pallas kernel here")


# speedup vs baseline: 1.5639x; 1.5639x over previous
import functools

import jax
import jax.numpy as jnp
from jax import lax
from jax.experimental import pallas as pl
from jax.experimental.pallas import tpu as pltpu
from jax.experimental.pallas import tpu_sc as plsc

B = 4096
EMB = 64
L = 50
LPAD = 56
NC = 2
NS = 16
NW = NC * NS
BPW = B // NW
SCALE = float(1.0 / 50.0)
BLK = 512


def _sc_embed(an_i, tu_i, tn_i, al_i, gn_i, name_i, tuc_i,
              an_t, tu_t, tn_t, al_t, gn_t, name_t, tuc_t):
  mesh = plsc.VectorSubcoreMesh(core_axis_name="c", subcore_axis_name="s")
  o = jax.ShapeDtypeStruct((B, EMB), jnp.float32)

  @functools.partial(
      pl.kernel, mesh=mesh,
      out_type=[o] * 7,
      compiler_params=pltpu.CompilerParams(use_tc_tiling_on_sc=False),
      scratch_types=[
          pltpu.VMEM((BPW, LPAD), jnp.int32),
          pltpu.VMEM((LPAD, EMB), jnp.float32),
          pltpu.VMEM((LPAD, EMB), jnp.float32),
          pltpu.VMEM((BPW, EMB), jnp.float32),
          pltpu.VMEM((BPW,), jnp.int32),
          pltpu.SemaphoreType.DMA,
          pltpu.SemaphoreType.DMA,
      ])
  def body(an_ir, tu_ir, tn_ir, al_ir, gn_ir, name_ir, tuc_ir,
           an_tr, tu_tr, tn_tr, al_tr, gn_tr, name_tr, tuc_tr,
           o_an, o_tu, o_tn, o_al, o_gn, o_name, o_tuc,
           idxp, buf_a, buf_b, outv, idx1, sem_a, sem_b):
    wid = lax.axis_index("s") * NC + lax.axis_index("c")
    base = wid * BPW

    for ir, tr, orf in ((name_ir, name_tr, o_name), (tuc_ir, tuc_tr, o_tuc)):
      pltpu.sync_copy(ir.at[pl.ds(base, BPW)], idx1)
      pltpu.async_copy(tr.at[idx1], outv, sem_a).wait()
      pltpu.sync_copy(outv, orf.at[pl.ds(base, BPW), :])

    for ir, tr, orf in ((an_ir, an_tr, o_an), (tu_ir, tu_tr, o_tu),
                        (tn_ir, tn_tr, o_tn), (al_ir, al_tr, o_al),
                        (gn_ir, gn_tr, o_gn)):
      pltpu.sync_copy(ir.at[pl.ds(base, BPW), :], idxp)
      pltpu.async_copy(tr.at[idxp.at[0, :]], buf_a, sem_a)
      pltpu.async_copy(tr.at[idxp.at[1, :]], buf_b, sem_b)

      @pl.loop(0, BPW, step=2)
      def _(b):
        for p, buf, sem in ((0, buf_a, sem_a), (1, buf_b, sem_b)):
          bb = b + p
          pltpu.make_async_copy(tr.at[idxp.at[0, :]], buf, sem).wait()

          def accl(i, accs, buf=buf):
            a = list(accs)
            for j in range(5):
              r = i * 5 + j
              for c in range(4):
                a[c] = a[c] + buf[r, pl.ds(16 * c, 16)]
            return tuple(a)

          z = jnp.zeros((16,), jnp.float32)
          accs = lax.fori_loop(0, 10, accl, (z, z, z, z))
          for c in range(4):
            outv[bb, pl.ds(16 * c, 16)] = accs[c] * SCALE

          @pl.when(bb + 2 < BPW)
          def _():
            pltpu.async_copy(tr.at[idxp.at[bb + 2, :]], buf, sem)

      pltpu.sync_copy(outv, orf.at[pl.ds(base, BPW), :])

  return body(an_i, tu_i, tn_i, al_i, gn_i, name_i, tuc_i,
              an_t, tu_t, tn_t, al_t, gn_t, name_t, tuc_t)


def _tc_body(name_r, collab_ir, tuc_r, an_r, tu_r, tn_r, dur_ir, al_r,
             ap_ir, fo_ir, tp_ir, gn_r,
             ct_r, dt_r, apt_r, fot_r, tpt_r,
             w1_r, b1_r, w2_r, b2_r, w3_r, b3_r, o_r):
  f32 = jnp.float32

  def bucket(ir, tr, v_size):
    idx = ir[...]
    acc = jnp.zeros((BLK, EMB), f32)
    for v in range(v_size):
      cnt = jnp.sum((idx == v).astype(f32), axis=1, keepdims=True)
      acc = acc + cnt * tr[v:v + 1, :]
    return acc * SCALE

  def single_small(ir, tr, v_size):
    idx = ir[...]
    acc = jnp.zeros((BLK, EMB), f32)
    for v in range(v_size):
      acc = acc + (idx == v).astype(f32) * tr[v:v + 1, :]
    return acc

  embs = [
      name_r[...],
      single_small(collab_ir, ct_r, 4),
      tuc_r[...],
      an_r[...],
      tu_r[...],
      tn_r[...],
      bucket(dur_ir, dt_r, 21),
      al_r[...],
      bucket(ap_ir, apt_r, 21),
      bucket(fo_ir, fot_r, 21),
      bucket(tp_ir, tpt_r, 21),
      gn_r[...],
  ]
  acc = None
  for f, e in enumerate(embs):
    t = jnp.dot(e, w1_r[f * EMB:(f + 1) * EMB, :], preferred_element_type=f32)
    acc = t if acc is None else acc + t
  h1 = jnp.maximum(acc + b1_r[...], 0.0)
  h2 = jnp.maximum(jnp.dot(h1, w2_r[...], preferred_element_type=f32)
                   + b2_r[...], 0.0)
  o_r[...] = jnp.dot(h2, w3_r[...], preferred_element_type=f32) + b3_r[...]


def _tc_mlp(name_e, collab_i2, tuc_e, an_e, tu_e, tn_e, dur_i, al_e,
            ap_i, fo_i, tp_i, gn_e,
            collab_t, dur_t, ap_t, fo_t, tp_t,
            w1, b1r, w2, b2r, w3, b3r):
  grid = (B // BLK,)
  blk = lambda s: pl.BlockSpec((BLK, s), lambda i: (i, 0))
  full = lambda a: pl.BlockSpec(a.shape, lambda i: (0, 0))
  in_specs = [
      blk(EMB), blk(1), blk(EMB), blk(EMB), blk(EMB), blk(EMB),
      blk(L), blk(EMB), blk(L), blk(L), blk(L), blk(EMB),
      full(collab_t), full(dur_t), full(ap_t), full(fo_t), full(tp_t),
      full(w1), full(b1r), full(w2), full(b2r), full(w3), full(b3r),
  ]
  return pl.pallas_call(
      _tc_body,
      grid=grid,
      in_specs=in_specs,
      out_specs=pl.BlockSpec((BLK, 128), lambda i: (i, 0)),
      out_shape=jax.ShapeDtypeStruct((B, 128), jnp.float32),
  )(name_e, collab_i2, tuc_e, an_e, tu_e, tn_e, dur_i, al_e,
    ap_i, fo_i, tp_i, gn_e, collab_t, dur_t, ap_t, fo_t, tp_t,
    w1, b1r, w2, b2r, w3, b3r)


def kernel(name_idx, collaborative_idx, track_uri_can_idx, artist_name_pl_idx,
           track_uri_pl_idx, track_name_pl_idx, duration_ms_pl_idx,
           album_name_pl_idx, artist_pop_pl_idx, artists_followers_pl_idx,
           track_pop_pl_idx, artist_genres_pl_idx,
           name_table, collab_table, track_uri_can_table, artist_name_table,
           track_uri_pl_table, track_name_table, duration_table, album_table,
           artist_pop_table, followers_table, track_pop_table, genres_table,
           W1, b1, W2, b2, W3, b3):
  pad6 = lambda x: jnp.pad(x, ((0, 0), (0, LPAD - L)))
  an_e, tu_e, tn_e, al_e, gn_e, name_e, tuc_e = _sc_embed(
      pad6(artist_name_pl_idx), pad6(track_uri_pl_idx),
      pad6(track_name_pl_idx), pad6(album_name_pl_idx),
      pad6(artist_genres_pl_idx), name_idx, track_uri_can_idx,
      artist_name_table, track_uri_pl_table, track_name_table, album_table,
      genres_table, name_table, track_uri_can_table)
  return _tc_mlp(
      name_e, collaborative_idx.reshape(B, 1), tuc_e, an_e, tu_e, tn_e,
      duration_ms_pl_idx, al_e, artist_pop_pl_idx, artists_followers_pl_idx,
      track_pop_pl_idx, gn_e,
      collab_table, duration_table, artist_pop_table, followers_table,
      track_pop_table,
      W1, b1.reshape(1, -1), W2, b2.reshape(1, -1), W3, b3.reshape(1, -1))


# speedup vs baseline: 2.6274x; 1.6800x over previous
import functools

import jax
import jax.numpy as jnp
from jax import lax
from jax.experimental import pallas as pl
from jax.experimental.pallas import tpu as pltpu
from jax.experimental.pallas import tpu_sc as plsc

B = 4096
EMB = 64
L = 50
LPAD = 56
NC = 2
NS = 16
NW = NC * NS
BPW = B // NW
SCALE = float(1.0 / 50.0)
BLK = 512


def _sc_embed(an_i, tu_i, tn_i, al_i, gn_i, name_i, tuc_i,
              an_t, tu_t, tn_t, al_t, gn_t, name_t, tuc_t):
  mesh = plsc.VectorSubcoreMesh(core_axis_name="c", subcore_axis_name="s")
  o = jax.ShapeDtypeStruct((B, EMB), jnp.float32)

  GRP = 2 * LPAD
  NGRP = BPW // 2
  NBUF = 4

  @functools.partial(
      pl.kernel, mesh=mesh,
      out_type=[o] * 7,
      compiler_params=pltpu.CompilerParams(use_tc_tiling_on_sc=False),
      scratch_types=[
          pltpu.VMEM((BPW * LPAD,), jnp.int32),
          [pltpu.VMEM((GRP, EMB), jnp.float32) for _ in range(NBUF)],
          pltpu.VMEM((BPW, EMB), jnp.float32),
          pltpu.VMEM((BPW,), jnp.int32),
          [pltpu.SemaphoreType.DMA for _ in range(NBUF)],
      ])
  def body(an_ir, tu_ir, tn_ir, al_ir, gn_ir, name_ir, tuc_ir,
           an_tr, tu_tr, tn_tr, al_tr, gn_tr, name_tr, tuc_tr,
           o_an, o_tu, o_tn, o_al, o_gn, o_name, o_tuc,
           idxp, bufs, outv, idx1, sems):
    wid = lax.axis_index("s") * NC + lax.axis_index("c")
    base = wid * BPW

    for ir, tr, orf in ((name_ir, name_tr, o_name), (tuc_ir, tuc_tr, o_tuc)):
      pltpu.sync_copy(ir.at[pl.ds(base, BPW)], idx1)
      pltpu.async_copy(tr.at[idx1], outv, sems[0]).wait()
      pltpu.sync_copy(outv, orf.at[pl.ds(base, BPW), :])

    def accum_row(buf, row0, bb):
      def accl(i, accs, buf=buf, row0=row0):
        a = list(accs)
        for j in range(5):
          r = row0 + i * 5 + j
          for c in range(4):
            a[c] = a[c] + buf[r, pl.ds(16 * c, 16)]
        return tuple(a)

      z = jnp.zeros((16,), jnp.float32)
      accs = lax.fori_loop(0, 10, accl, (z, z, z, z))
      for c in range(4):
        outv[bb, pl.ds(16 * c, 16)] = accs[c] * SCALE

    for ir, tr, orf in ((an_ir, an_tr, o_an), (tu_ir, tu_tr, o_tu),
                        (tn_ir, tn_tr, o_tn), (al_ir, al_tr, o_al),
                        (gn_ir, gn_tr, o_gn)):
      pltpu.sync_copy(ir.at[pl.ds(base * LPAD, BPW * LPAD)], idxp)
      for p in range(NBUF):
        pltpu.async_copy(tr.at[idxp.at[pl.ds(p * GRP, GRP)]], bufs[p], sems[p])

      @pl.loop(0, NGRP, step=NBUF)
      def _(g):
        for p in range(NBUF):
          gg = g + p
          buf, sem = bufs[p], sems[p]
          pltpu.make_async_copy(tr.at[idxp.at[pl.ds(0, GRP)]], buf, sem).wait()
          accum_row(buf, 0, 2 * gg)
          accum_row(buf, LPAD, 2 * gg + 1)

          @pl.when(gg + NBUF < NGRP)
          def _():
            pltpu.async_copy(
                tr.at[idxp.at[pl.ds((gg + NBUF) * GRP, GRP)]], buf, sem)

      pltpu.sync_copy(outv, orf.at[pl.ds(base, BPW), :])

  return body(an_i, tu_i, tn_i, al_i, gn_i, name_i, tuc_i,
              an_t, tu_t, tn_t, al_t, gn_t, name_t, tuc_t)


def _tc_body(name_r, collab_ir, tuc_r, an_r, tu_r, tn_r, dur_ir, al_r,
             ap_ir, fo_ir, tp_ir, gn_r,
             ct_r, dt_r, apt_r, fot_r, tpt_r,
             w1_r, b1_r, w2_r, b2_r, w3_r, b3_r, o_r):
  f32 = jnp.float32

  def bucket(ir, tr, v_size):
    idx = ir[...]
    acc = jnp.zeros((BLK, EMB), f32)
    for v in range(v_size):
      cnt = jnp.sum((idx == v).astype(f32), axis=1, keepdims=True)
      acc = acc + cnt * tr[v:v + 1, :]
    return acc * SCALE

  def single_small(ir, tr, v_size):
    idx = ir[...]
    acc = jnp.zeros((BLK, EMB), f32)
    for v in range(v_size):
      acc = acc + (idx == v).astype(f32) * tr[v:v + 1, :]
    return acc

  embs = [
      name_r[...],
      single_small(collab_ir, ct_r, 4),
      tuc_r[...],
      an_r[...],
      tu_r[...],
      tn_r[...],
      bucket(dur_ir, dt_r, 21),
      al_r[...],
      bucket(ap_ir, apt_r, 21),
      bucket(fo_ir, fot_r, 21),
      bucket(tp_ir, tpt_r, 21),
      gn_r[...],
  ]
  acc = None
  for f, e in enumerate(embs):
    t = jnp.dot(e, w1_r[f * EMB:(f + 1) * EMB, :], preferred_element_type=f32)
    acc = t if acc is None else acc + t
  h1 = jnp.maximum(acc + b1_r[...], 0.0)
  h2 = jnp.maximum(jnp.dot(h1, w2_r[...], preferred_element_type=f32)
                   + b2_r[...], 0.0)
  o_r[...] = jnp.dot(h2, w3_r[...], preferred_element_type=f32) + b3_r[...]


def _tc_mlp(name_e, collab_i2, tuc_e, an_e, tu_e, tn_e, dur_i, al_e,
            ap_i, fo_i, tp_i, gn_e,
            collab_t, dur_t, ap_t, fo_t, tp_t,
            w1, b1r, w2, b2r, w3, b3r):
  grid = (B // BLK,)
  blk = lambda s: pl.BlockSpec((BLK, s), lambda i: (i, 0))
  full = lambda a: pl.BlockSpec(a.shape, lambda i: (0, 0))
  in_specs = [
      blk(EMB), blk(1), blk(EMB), blk(EMB), blk(EMB), blk(EMB),
      blk(L), blk(EMB), blk(L), blk(L), blk(L), blk(EMB),
      full(collab_t), full(dur_t), full(ap_t), full(fo_t), full(tp_t),
      full(w1), full(b1r), full(w2), full(b2r), full(w3), full(b3r),
  ]
  return pl.pallas_call(
      _tc_body,
      grid=grid,
      in_specs=in_specs,
      out_specs=pl.BlockSpec((BLK, 128), lambda i: (i, 0)),
      out_shape=jax.ShapeDtypeStruct((B, 128), jnp.float32),
  )(name_e, collab_i2, tuc_e, an_e, tu_e, tn_e, dur_i, al_e,
    ap_i, fo_i, tp_i, gn_e, collab_t, dur_t, ap_t, fo_t, tp_t,
    w1, b1r, w2, b2r, w3, b3r)


def kernel(name_idx, collaborative_idx, track_uri_can_idx, artist_name_pl_idx,
           track_uri_pl_idx, track_name_pl_idx, duration_ms_pl_idx,
           album_name_pl_idx, artist_pop_pl_idx, artists_followers_pl_idx,
           track_pop_pl_idx, artist_genres_pl_idx,
           name_table, collab_table, track_uri_can_table, artist_name_table,
           track_uri_pl_table, track_name_table, duration_table, album_table,
           artist_pop_table, followers_table, track_pop_table, genres_table,
           W1, b1, W2, b2, W3, b3):
  pad6 = lambda x: jnp.concatenate([x, x[:, :LPAD - L]], axis=1).reshape(-1)
  an_e, tu_e, tn_e, al_e, gn_e, name_e, tuc_e = _sc_embed(
      pad6(artist_name_pl_idx), pad6(track_uri_pl_idx),
      pad6(track_name_pl_idx), pad6(album_name_pl_idx),
      pad6(artist_genres_pl_idx), name_idx, track_uri_can_idx,
      artist_name_table, track_uri_pl_table, track_name_table, album_table,
      genres_table, name_table, track_uri_can_table)
  return _tc_mlp(
      name_e, collaborative_idx.reshape(B, 1), tuc_e, an_e, tu_e, tn_e,
      duration_ms_pl_idx, al_e, artist_pop_pl_idx, artists_followers_pl_idx,
      track_pop_pl_idx, gn_e,
      collab_table, duration_table, artist_pop_table, followers_table,
      track_pop_table,
      W1, b1.reshape(1, -1), W2, b2.reshape(1, -1), W3, b3.reshape(1, -1))
